# baseline (speedup 1.0000x reference)
import functools

import jax
import jax.numpy as jnp
from jax import lax
from jax.experimental import pallas as pl
from jax.experimental.pallas import tpu as pltpu
from jax.experimental.pallas import tpu_sc as plsc

B = 1024
L = 200
EMBED_DIM = 64
NUM_ENT_P1 = 1000001
PAD_ROWS = 1000008
PAD_COLS = 128

NUM_CORES = 2
NUM_SUBCORES = 16
NUM_WORKERS = NUM_CORES * NUM_SUBCORES
LANES = 16

BBLK = 128
NUM_BLOCKS = L * 2 * (B // BBLK)
BLOCKS_PER_WORKER = NUM_BLOCKS // NUM_WORKERS


def _gather_native(trip_t, tpad):
    mesh = plsc.VectorSubcoreMesh(core_axis_name="c", subcore_axis_name="s")

    @functools.partial(
        pl.kernel,
        mesh=mesh,
        out_type=jax.ShapeDtypeStruct((L, 2, EMBED_DIM, B), jnp.float32),
        compiler_params=pltpu.CompilerParams(
            needs_layout_passes=False, use_tc_tiling_on_sc=True
        ),
        scratch_types=[
            pltpu.VMEM((BBLK,), jnp.int32),
            pltpu.VMEM((BBLK, PAD_COLS), jnp.float32),
            pltpu.VMEM((EMBED_DIM, BBLK), jnp.float32),
            pltpu.SemaphoreType.DMA,
        ],
    )
    def k(trip_hbm, tpad_hbm, o2_hbm, idx_v, g_v, o_v, sem):
        wid = lax.axis_index("s") * NUM_CORES + lax.axis_index("c")
        lane = lax.iota(jnp.int32, LANES)

        def blk(t, _):
            bid = wid * BLOCKS_PER_WORKER + t
            l = bid // 16
            j = (bid // 8) & 1
            bb = bid & 7
            pltpu.sync_copy(
                trip_hbm.at[2 * j, l, pl.ds(BBLK * bb, BBLK)], idx_v
            )
            pltpu.async_copy(tpad_hbm.at[idx_v], g_v, sem).wait()
            for c in range(BBLK // LANES):
                e16 = c * LANES + lane
                for d in range(EMBED_DIM):
                    d16 = jnp.full((LANES,), d, jnp.int32)
                    o_v[d, pl.ds(c * LANES, LANES)] = plsc.load_gather(
                        g_v, [e16, d16]
                    )
            pltpu.sync_copy(
                o_v, o2_hbm.at[l, j, :, pl.ds(BBLK * bb, BBLK)]
            )
            return _

        lax.fori_loop(0, BLOCKS_PER_WORKER, blk, None)

    return k(trip_t, tpad)


def kernel(triples, emb_table):
    tpad = jnp.pad(emb_table, ((0, PAD_ROWS - NUM_ENT_P1), (0, PAD_COLS - EMBED_DIM)))
    trip_t = jnp.transpose(triples, (2, 1, 0))
    o2 = _gather_native(trip_t, tpad)
    return jnp.transpose(o2, (3, 0, 1, 2))
